# baseline (speedup 1.0000x reference)
import functools
import math

import jax
import jax.numpy as jnp
from jax import lax
from jax.experimental import pallas as pl
from jax.experimental.pallas import tpu as pltpu
from jax.experimental.pallas import tpu_sc as plsc

N = 10000
E = 320000
IDIM = 128
HDIM = 200
ODIM = 200
OLEN = 512

NB = 128
HALF = 128
DP = 2 * HALF

NP0 = 10240
E2 = E + N
E2P = 331776
EP = 323584


def _rup(x, m):
    return (x + m - 1) // m * m


def _mm_body(prologue, hcat_ref, w_ref, asd_ref, b_ref, m_ref,
             xpa_ref, xpb_ref, ssd_ref, bm_ref):
    h = hcat_ref[...]
    if prologue:
        h = jnp.maximum(h + b_ref[...], 0.0) * m_ref[...]
    res = jnp.dot(h, w_ref[...], preferred_element_type=jnp.float32)
    z12 = jnp.zeros((res.shape[0], 28), jnp.float32)
    xpa_ref[...] = jnp.concatenate([res[:, :100], z12], axis=1)
    xpb_ref[...] = jnp.concatenate([res[:, 100:200], z12], axis=1)
    ssd = jnp.dot(res, asd_ref[...], preferred_element_type=jnp.float32)
    ssd_ref[...] = ssd
    bm_ref[...] = jnp.max(ssd[:, :1], axis=0, keepdims=True)[None]


def _mm_call(hcat, w, asd, b, m, prologue):
    np_, dpin = hcat.shape
    grid = np_ // 256
    return pl.pallas_call(
        functools.partial(_mm_body, prologue),
        grid=(grid,),
        in_specs=[
            pl.BlockSpec((256, dpin), lambda i: (i, 0)),
            pl.BlockSpec((dpin, 256), lambda i: (0, 0)),
            pl.BlockSpec((256, 2), lambda i: (0, 0)),
            pl.BlockSpec((1, dpin), lambda i: (0, 0)),
            pl.BlockSpec((256, dpin), lambda i: (i, 0)),
        ],
        out_specs=[
            pl.BlockSpec((256, HALF), lambda i: (i, 0)),
            pl.BlockSpec((256, HALF), lambda i: (i, 0)),
            pl.BlockSpec((256, 2), lambda i: (i, 0)),
            pl.BlockSpec((1, 1, 1), lambda i: (i, 0, 0)),
        ],
        out_shape=[
            jax.ShapeDtypeStruct((np_, HALF), jnp.float32),
            jax.ShapeDtypeStruct((np_, HALF), jnp.float32),
            jax.ShapeDtypeStruct((np_, 2), jnp.float32),
            jax.ShapeDtypeStruct((grid, 1, 1), jnp.float32),
        ],
    )(hcat, w, asd, b, m)


def _lrelu(v):
    return jnp.where(v >= 0, v, 0.2 * v)


def _fin1_body(t0_ref, t1_ref, ssd_ref, c_ref, out_ref):
    t = t0_ref[0, :] + t1_ref[0, :]
    t = jnp.maximum(t, 1e-30)
    out_ref[...] = _lrelu(jnp.log(t) + c_ref[0, 0] + ssd_ref[:, 1])[None, :]


def _fin1_call(t0, t1, ssd, c):
    np_ = ssd.shape[0]
    return pl.pallas_call(
        _fin1_body,
        out_shape=jax.ShapeDtypeStruct((1, np_), jnp.float32),
    )(t0.reshape(1, np_), t1.reshape(1, np_), ssd, c)


def _fin2_body(d0_ref, d1_ref, out_ref):
    out_ref[...] = 1.0 / (d0_ref[0, :] + d1_ref[0, :] + 1e-16)[None, :]


def _fin2_call(d0, d1):
    np_ = d0.shape[0]
    return pl.pallas_call(
        _fin2_body,
        out_shape=jax.ShapeDtypeStruct((1, np_), jnp.float32),
    )(d0.reshape(1, np_), d1.reshape(1, np_))


def _h3z_body(agga_ref, aggb_ref, ba_ref, bb_ref, ma_ref, mb_ref, wp_ref,
              ha_ref, hb_ref, z_ref):
    ha = jnp.maximum(agga_ref[...] + ba_ref[...], 0.0) * ma_ref[...]
    hb = jnp.maximum(aggb_ref[...] + bb_ref[...], 0.0) * mb_ref[...]
    ha_ref[...] = ha
    hb_ref[...] = hb
    hcat = jnp.concatenate([ha, hb], axis=1)
    z = jnp.dot(hcat, wp_ref[...], preferred_element_type=jnp.float32)
    z_ref[...] = z[:, :2]


def _h3z_call(agga, aggb, ba, bb, ma, mb, wp):
    np_ = agga.shape[0]
    grid = np_ // 256
    return pl.pallas_call(
        _h3z_body,
        grid=(grid,),
        in_specs=[
            pl.BlockSpec((256, HALF), lambda i: (i, 0)),
            pl.BlockSpec((256, HALF), lambda i: (i, 0)),
            pl.BlockSpec((1, HALF), lambda i: (0, 0)),
            pl.BlockSpec((1, HALF), lambda i: (0, 0)),
            pl.BlockSpec((256, HALF), lambda i: (i, 0)),
            pl.BlockSpec((256, HALF), lambda i: (i, 0)),
            pl.BlockSpec((DP, 128), lambda i: (0, 0)),
        ],
        out_specs=[
            pl.BlockSpec((256, HALF), lambda i: (i, 0)),
            pl.BlockSpec((256, HALF), lambda i: (i, 0)),
            pl.BlockSpec((256, 2), lambda i: (i, 0)),
        ],
        out_shape=[
            jax.ShapeDtypeStruct((np_, HALF), jnp.float32),
            jax.ShapeDtypeStruct((np_, HALF), jnp.float32),
            jax.ShapeDtypeStruct((np_, 2), jnp.float32),
        ],
    )(agga, aggb, ba, bb, ma, mb, wp)


def _score_body(nt, sagg0_ref, sagg1_ref, z2_ref, bp_ref, score_ref, ts_ref):
    np_ = z2_ref.shape[1]
    s = sagg0_ref[0, :] + sagg1_ref[0, :] + bp_ref[0, 0] + z2_ref[0, :]
    col = lax.broadcasted_iota(jnp.int32, (1, np_), 1)
    s = jnp.where(col < nt, s[None, :], -jnp.inf)
    score_ref[...] = s
    ts_ref[...] = jnp.tanh(s)


def _score_call(sagg0, sagg1, z2, bp, nt):
    np_ = z2.shape[1]
    return pl.pallas_call(
        functools.partial(_score_body, nt),
        out_shape=[
            jax.ShapeDtypeStruct((1, np_), jnp.float32),
            jax.ShapeDtypeStruct((1, np_), jnp.float32),
        ],
    )(sagg0.reshape(1, np_), sagg1.reshape(1, np_), z2, bp)


def _rank_body(s2d_ref, rank_ref):
    i = pl.program_id(0)
    nj = s2d_ref.shape[0]
    si = jnp.reshape(s2d_ref[i, :], (256, 1))
    ii = i * 256 + lax.broadcasted_iota(jnp.int32, (256, 1), 0)

    def jloop(j, acc):
        sj = jnp.reshape(s2d_ref[j, :], (1, 256))
        jj = j * 256 + lax.broadcasted_iota(jnp.int32, (1, 256), 1)
        gt = (sj > si).astype(jnp.float32)
        eq = jnp.logical_and(sj == si, jj < ii).astype(jnp.float32)
        return acc + jnp.sum(gt + eq, axis=1, keepdims=True)

    acc = lax.fori_loop(0, nj, jloop, jnp.zeros((256, 1), jnp.float32))
    rank_ref[...] = acc.astype(jnp.int32)


def _tsmul_body(ha_ref, hb_ref, z_ref, ts_ref, hsa_ref, hsb_ref, zs_ref):
    t = ts_ref[...]
    hsa_ref[...] = ha_ref[...] * t
    hsb_ref[...] = hb_ref[...] * t
    zs_ref[...] = z_ref[...] * t


def _tsmul_call(ha, hb, z12, tscol):
    np_ = ha.shape[0]
    grid = np_ // 256
    return pl.pallas_call(
        _tsmul_body,
        grid=(grid,),
        in_specs=[
            pl.BlockSpec((256, HALF), lambda i: (i, 0)),
            pl.BlockSpec((256, HALF), lambda i: (i, 0)),
            pl.BlockSpec((256, 2), lambda i: (i, 0)),
            pl.BlockSpec((256, 1), lambda i: (i, 0)),
        ],
        out_specs=[
            pl.BlockSpec((256, HALF), lambda i: (i, 0)),
            pl.BlockSpec((256, HALF), lambda i: (i, 0)),
            pl.BlockSpec((256, 2), lambda i: (i, 0)),
        ],
        out_shape=[
            jax.ShapeDtypeStruct((np_, HALF), jnp.float32),
            jax.ShapeDtypeStruct((np_, HALF), jnp.float32),
            jax.ShapeDtypeStruct((np_, 2), jnp.float32),
        ],
    )(ha, hb, z12, tscol)


def _rank_call(s2d):
    np_ = s2d.shape[0] * 256
    return pl.pallas_call(
        _rank_body,
        grid=(s2d.shape[0],),
        in_specs=[pl.BlockSpec((s2d.shape[0], 256), lambda i: (0, 0))],
        out_specs=pl.BlockSpec((256, 1), lambda i: (i, 0)),
        out_shape=jax.ShapeDtypeStruct((np_, 1), jnp.int32),
    )(s2d)


_MESH_CACHE = []


_SC_PARAMS = pltpu.CompilerParams(needs_layout_passes=False)


def _mesh():
    if not _MESH_CACHE:
        _MESH_CACHE.append(plsc.VectorSubcoreMesh(
            core_axis_name="c", subcore_axis_name="s"))
    return _MESH_CACHE[0]


def _wid():
    c = lax.axis_index("c")
    s = lax.axis_index("s")
    return c, s, s * 2 + c


def _zero_acc1(acc, zb, sid, pw):
    def zi(i, _):
        zb[pl.ds(i * 16, 16)] = jnp.zeros((16,), jnp.float32)
        return 0
    lax.fori_loop(0, pw // 16, zi, 0)
    pltpu.sync_copy(zb, acc.at[pl.ds(sid * pw, pw)])


def _wb_acc1(acc, out, sid, pw, zb):
    pltpu.sync_copy(acc.at[pl.ds(sid * pw, pw)], zb)
    pltpu.sync_copy(zb, out.at[pl.ds(sid * pw, pw)])


def _s1_body(np_, gs, gd, ss, c_h, out0, out1, idx_s, idx_d, val, tbl, c_v, acc, zb):
    c, sid, w = _wid()
    pw = np_ // 16
    _zero_acc1(acc, zb, sid, pw)
    pltpu.sync_copy(ss, tbl)
    pltpu.sync_copy(c_h, c_v)
    plsc.subcore_barrier()
    epw = E2P // 32
    base0 = w * epw
    cmax = c_v[pl.ds(0, 16)][0]

    def chunk(j, _):
        base = base0 + j * NB
        pltpu.sync_copy(gs.at[pl.ds(base, NB)], idx_s)
        pltpu.sync_copy(gd.at[pl.ds(base, NB)], idx_d)

        def lane(v, _):
            sv = idx_s[pl.ds(v * 16, 16)]
            ssv = plsc.load_gather(tbl, [sv])
            eid = base + v * 16 + lax.iota(jnp.int32, 16)
            val[pl.ds(v * 16, 16)] = jnp.where(
                eid < E2, jnp.exp(ssv - cmax), 0.0)
            return 0

        lax.fori_loop(0, NB // 16, lane, 0)
        pltpu.sync_copy(val, acc.at[idx_d], add=True)
        return 0

    lax.fori_loop(0, epw // NB, chunk, 0)
    plsc.subcore_barrier()

    @pl.when(c == 0)
    def _():
        _wb_acc1(acc, out0, sid, pw, zb)

    @pl.when(c == 1)
    def _():
        _wb_acc1(acc, out1, sid, pw, zb)


def _s1_call(np_, gs, gd, ss, c_h):
    fn = pl.kernel(
        functools.partial(_s1_body, np_),
        out_type=[jax.ShapeDtypeStruct((np_,), jnp.float32),
                  jax.ShapeDtypeStruct((np_,), jnp.float32)],
        mesh=_mesh(),
        compiler_params=_SC_PARAMS,
        scratch_types=[
            pltpu.VMEM((NB,), jnp.int32),
            pltpu.VMEM((NB,), jnp.int32),
            pltpu.VMEM((NB,), jnp.float32),
            pltpu.VMEM((np_,), jnp.float32),
            pltpu.VMEM((16,), jnp.float32),
            pltpu.VMEM_SHARED((np_,), jnp.float32),
            pltpu.VMEM((np_ // 16,), jnp.float32),
        ],
    )
    return fn(gs, gd, ss, c_h)


def _s2_body(np_, gs, gd, ss, sd, mt, d_out0, d_out1, ex_out,
             idx_s, idx_d, val, tbl_s, tbl_d, tbl_m, acc, zb):
    c, sid, w = _wid()
    pw = np_ // 16
    _zero_acc1(acc, zb, sid, pw)
    pltpu.sync_copy(ss, tbl_s)
    pltpu.sync_copy(sd, tbl_d)
    pltpu.sync_copy(mt, tbl_m)
    plsc.subcore_barrier()
    epw = E2P // 32
    base0 = w * epw

    def chunk(j, _):
        base = base0 + j * NB
        pltpu.sync_copy(gs.at[pl.ds(base, NB)], idx_s)
        pltpu.sync_copy(gd.at[pl.ds(base, NB)], idx_d)

        def lane(v, _):
            sv = idx_s[pl.ds(v * 16, 16)]
            dv = idx_d[pl.ds(v * 16, 16)]
            al = _lrelu(plsc.load_gather(tbl_s, [sv]) +
                        plsc.load_gather(tbl_d, [dv]))
            mv = plsc.load_gather(tbl_m, [dv])
            eid = base + v * 16 + lax.iota(jnp.int32, 16)
            val[pl.ds(v * 16, 16)] = jnp.where(
                eid < E2, jnp.exp(al - mv), 0.0)
            return 0

        lax.fori_loop(0, NB // 16, lane, 0)
        pltpu.sync_copy(val, ex_out.at[pl.ds(base, NB)])
        pltpu.sync_copy(val, acc.at[idx_d], add=True)
        return 0

    lax.fori_loop(0, epw // NB, chunk, 0)
    plsc.subcore_barrier()

    @pl.when(c == 0)
    def _():
        _wb_acc1(acc, d_out0, sid, pw, zb)

    @pl.when(c == 1)
    def _():
        _wb_acc1(acc, d_out1, sid, pw, zb)


def _s2_call(np_, gs, gd, ss, sd, mt):
    fn = pl.kernel(
        functools.partial(_s2_body, np_),
        out_type=[
            jax.ShapeDtypeStruct((np_,), jnp.float32),
            jax.ShapeDtypeStruct((np_,), jnp.float32),
            jax.ShapeDtypeStruct((E2P,), jnp.float32),
        ],
        mesh=_mesh(),
        compiler_params=_SC_PARAMS,
        scratch_types=[
            pltpu.VMEM((NB,), jnp.int32),
            pltpu.VMEM((NB,), jnp.int32),
            pltpu.VMEM((NB,), jnp.float32),
            pltpu.VMEM((np_,), jnp.float32),
            pltpu.VMEM((np_,), jnp.float32),
            pltpu.VMEM((np_,), jnp.float32),
            pltpu.VMEM_SHARED((np_,), jnp.float32),
            pltpu.VMEM((np_ // 16,), jnp.float32),
        ],
    )
    return fn(gs, gd, ss, sd, mt)


def _msg_body(np_, gs, gd, ex, invd, xpa, xpb, agga, aggb,
              idx_s, idx_d, exv, cfv, rows, tbl_i, zb, acc, sem):
    c, sid, _ = _wid()
    prow = np_ // 16

    def zi(r, _):
        def zf(f, _):
            zb[r, pl.ds(f * 16, 16)] = jnp.zeros((16,), jnp.float32)
            return 0
        lax.fori_loop(0, 8, zf, 0)
        return 0
    lax.fori_loop(0, 16, zi, 0)

    def zcp(j, _):
        pltpu.sync_copy(zb, acc.at[pl.ds(sid * prow + j * 16, 16), :])
        return 0
    lax.fori_loop(0, prow // 16, zcp, 0)
    pltpu.sync_copy(invd, tbl_i)
    plsc.subcore_barrier()

    epw = E2P // 16
    base0 = sid * epw

    def run(xp, agg):
        def chunk(j, _):
            base = base0 + j * NB
            pltpu.sync_copy(gs.at[pl.ds(base, NB)], idx_s)
            pltpu.sync_copy(gd.at[pl.ds(base, NB)], idx_d)
            pltpu.sync_copy(ex.at[pl.ds(base, NB)], exv)
            cp = pltpu.async_copy(xp.at[idx_s], rows, sem)

            def lane(v, _):
                dv = idx_d[pl.ds(v * 16, 16)]
                iv = plsc.load_gather(tbl_i, [dv])
                cfv[pl.ds(v * 16, 16)] = exv[pl.ds(v * 16, 16)] * iv
                return 0

            lax.fori_loop(0, NB // 16, lane, 0)
            cp.wait()

            def scale(g, _):
                cf16 = cfv[pl.ds(g * 16, 16)]
                for l in range(16):
                    r = g * 16 + l
                    cs = cf16[l]

                    def sf(f, _):
                        rows[r, pl.ds(f * 16, 16)] = (
                            rows[r, pl.ds(f * 16, 16)] * cs)
                        return 0
                    lax.fori_loop(0, 8, sf, 0)
                return 0

            lax.fori_loop(0, NB // 16, scale, 0)
            pltpu.sync_copy(rows, acc.at[idx_d], add=True)
            return 0

        lax.fori_loop(0, epw // NB, chunk, 0)
        plsc.subcore_barrier()
        pltpu.sync_copy(acc.at[pl.ds(sid * prow, prow), :],
                        agg.at[pl.ds(sid * prow, prow), :])

    @pl.when(c == 0)
    def _():
        run(xpa, agga)

    @pl.when(c == 1)
    def _():
        run(xpb, aggb)


def _msg_call(np_, gs, gd, ex, invd, xpa, xpb):
    fn = pl.kernel(
        functools.partial(_msg_body, np_),
        out_type=[
            jax.ShapeDtypeStruct((np_, HALF), jnp.float32),
            jax.ShapeDtypeStruct((np_, HALF), jnp.float32),
        ],
        mesh=_mesh(),
        compiler_params=_SC_PARAMS,
        scratch_types=[
            pltpu.VMEM((NB,), jnp.int32),
            pltpu.VMEM((NB,), jnp.int32),
            pltpu.VMEM((NB,), jnp.float32),
            pltpu.VMEM((NB,), jnp.float32),
            pltpu.VMEM((NB, HALF), jnp.float32),
            pltpu.VMEM((np_,), jnp.float32),
            pltpu.VMEM((16, HALF), jnp.float32),
            pltpu.VMEM_SHARED((np_, HALF), jnp.float32),
            pltpu.SemaphoreType.DMA,
        ],
    )
    return fn(gs, gd, ex, invd, xpa, xpb)


def _p1_body(np_, npprev, kprev, ps, pd, em, rank, z1,
             ps_o, pd_o, em_o, sagg0, sagg1,
             idx_s, idx_d, emv, val, tbl_r, tbl_z, acc, zb):
    c, sid, w = _wid()
    pw = np_ // 16
    _zero_acc1(acc, zb, sid, pw)
    pltpu.sync_copy(rank, tbl_r)
    pltpu.sync_copy(z1, tbl_z)
    plsc.subcore_barrier()
    epw = EP // 32
    base0 = w * epw

    def chunk(j, _):
        base = base0 + j * NB
        pltpu.sync_copy(ps.at[pl.ds(base, NB)], idx_s)
        pltpu.sync_copy(pd.at[pl.ds(base, NB)], idx_d)
        pltpu.sync_copy(em.at[pl.ds(base, NB)], emv)

        def lane(v, _):
            sv = idx_s[pl.ds(v * 16, 16)]
            dv = idx_d[pl.ds(v * 16, 16)]
            rs = plsc.load_gather(tbl_r, [sv])
            rd = plsc.load_gather(tbl_r, [dv])
            oks = rs < kprev
            okd = rd < kprev
            ps2 = jnp.where(oks, rs, 0)
            pd2 = jnp.where(okd, rd, 0)
            em2 = emv[pl.ds(v * 16, 16)] * jnp.where(
                jnp.logical_and(oks, okd), 1.0, 0.0)
            idx_s[pl.ds(v * 16, 16)] = ps2
            idx_d[pl.ds(v * 16, 16)] = pd2
            emv[pl.ds(v * 16, 16)] = em2
            val[pl.ds(v * 16, 16)] = plsc.load_gather(tbl_z, [ps2]) * em2
            return 0

        lax.fori_loop(0, NB // 16, lane, 0)
        pltpu.sync_copy(idx_s, ps_o.at[pl.ds(base, NB)])
        pltpu.sync_copy(idx_d, pd_o.at[pl.ds(base, NB)])
        pltpu.sync_copy(emv, em_o.at[pl.ds(base, NB)])
        pltpu.sync_copy(val, acc.at[idx_d], add=True)
        return 0

    lax.fori_loop(0, epw // NB, chunk, 0)
    plsc.subcore_barrier()

    @pl.when(c == 0)
    def _():
        _wb_acc1(acc, sagg0, sid, pw, zb)

    @pl.when(c == 1)
    def _():
        _wb_acc1(acc, sagg1, sid, pw, zb)


def _p1_call(np_, npprev, kprev, ps, pd, em, rank, z1):
    fn = pl.kernel(
        functools.partial(_p1_body, np_, npprev, kprev),
        out_type=[
            jax.ShapeDtypeStruct((EP,), jnp.int32),
            jax.ShapeDtypeStruct((EP,), jnp.int32),
            jax.ShapeDtypeStruct((EP,), jnp.float32),
            jax.ShapeDtypeStruct((np_,), jnp.float32),
            jax.ShapeDtypeStruct((np_,), jnp.float32),
        ],
        mesh=_mesh(),
        compiler_params=_SC_PARAMS,
        scratch_types=[
            pltpu.VMEM((NB,), jnp.int32),
            pltpu.VMEM((NB,), jnp.int32),
            pltpu.VMEM((NB,), jnp.float32),
            pltpu.VMEM((NB,), jnp.float32),
            pltpu.VMEM((npprev,), jnp.int32),
            pltpu.VMEM((np_,), jnp.float32),
            pltpu.VMEM_SHARED((np_,), jnp.float32),
            pltpu.VMEM((np_ // 16,), jnp.float32),
        ],
    )
    return fn(ps, pd, em, rank, z1)


def _p3_body(np_, k, kp, ha, hb, z1, z2, rank,
             hoa, hob, z1o, z2o,
             rowsa, rowsb, rkv, tgt, zav, zbv, sema, semb):
    c, sid, w = _wid()
    nch = np_ // 16

    def chunk(jj, _):
        cid = w + jj * 32

        @pl.when(cid < nch)
        def _():
            base = cid * 16
            pltpu.sync_copy(rank.at[pl.ds(base, 16)], rkv)
            cpa = pltpu.async_copy(ha.at[pl.ds(base, 16), :], rowsa, sema)
            cpb = pltpu.async_copy(hb.at[pl.ds(base, 16), :], rowsb, semb)
            pltpu.sync_copy(z1.at[pl.ds(base, 16)], zav)
            pltpu.sync_copy(z2.at[pl.ds(base, 16)], zbv)
            rk = rkv[pl.ds(0, 16)]
            tgt[pl.ds(0, 16)] = jnp.where(rk < k, rk, kp - 1)
            cpa.wait()
            cpb.wait()
            pltpu.sync_copy(rowsa, hoa.at[tgt])
            pltpu.sync_copy(rowsb, hob.at[tgt])
            pltpu.sync_copy(zav, z1o.at[tgt])
            pltpu.sync_copy(zbv, z2o.at[tgt])
        return 0

    lax.fori_loop(0, (nch + 31) // 32, chunk, 0)


def _p3_call(np_, k, kp, ha, hb, z1, z2, rank):
    fn = pl.kernel(
        functools.partial(_p3_body, np_, k, kp),
        out_type=[
            jax.ShapeDtypeStruct((kp, HALF), jnp.float32),
            jax.ShapeDtypeStruct((kp, HALF), jnp.float32),
            jax.ShapeDtypeStruct((kp,), jnp.float32),
            jax.ShapeDtypeStruct((kp,), jnp.float32),
        ],
        mesh=_mesh(),
        compiler_params=_SC_PARAMS,
        scratch_types=[
            pltpu.VMEM((16, HALF), jnp.float32),
            pltpu.VMEM((16, HALF), jnp.float32),
            pltpu.VMEM((16,), jnp.int32),
            pltpu.VMEM((16,), jnp.int32),
            pltpu.VMEM((16,), jnp.float32),
            pltpu.VMEM((16,), jnp.float32),
            pltpu.SemaphoreType.DMA,
            pltpu.SemaphoreType.DMA,
        ],
    )
    return fn(ha, hb, z1, z2, rank)


def _pad_rows(a, np_):
    return jnp.pad(a, ((0, np_ - a.shape[0]),) + ((0, 0),) * (a.ndim - 1))


def _split_pad_cols(v):
    za = jnp.pad(v[..., :100], ((0, 0),) * (v.ndim - 1) + ((0, 28),))
    zb = jnp.pad(v[..., 100:200], ((0, 0),) * (v.ndim - 1) + ((0, 28),))
    return za, zb


def _gat_layer(np_, gs, gd, hcat, w_p, asd_p, b_prev, m_prev, prologue):
    xpa, xpb, ssd, bm = _mm_call(hcat, w_p, asd_p, b_prev, m_prev, prologue)
    c = jnp.max(bm).reshape(1, 1)
    ss = ssd[:, 0]
    c16 = jnp.broadcast_to(jnp.max(bm), (16,))
    t0, t1 = _s1_call(np_, gs, gd, ss, c16)
    mt = _fin1_call(t0, t1, ssd, c)[0]
    d0, d1, ex = _s2_call(np_, gs, gd, ss, ssd[:, 1], mt)
    invd = _fin2_call(d0, d1)[0]
    agga, aggb = _msg_call(np_, gs, gd, ex, invd, xpa, xpb)
    return agga, aggb


def kernel(x, edge_index, W1, a_s1, a_d1, b1, W2, a_s2, a_d2, b2,
           W3, a_s3, a_d3, b3, Wp_rel, bp_rel, Wp_root):
    f32 = jnp.float32
    src = edge_index[0].astype(jnp.int32)
    dst = edge_index[1].astype(jnp.int32)
    loop = jnp.arange(N, dtype=jnp.int32)
    gs = jnp.pad(jnp.concatenate([src, loop]), (0, E2P - E2))
    gd = jnp.pad(jnp.concatenate([dst, loop]), (0, E2P - E2))

    dk = jax.random.key(42)
    masks = []
    for i in range(3):
        keep = jax.random.bernoulli(jax.random.fold_in(dk, i), 0.8, (N, 200))
        masks.append(keep.astype(f32) * 1.25)

    hcat1 = _pad_rows(x, NP0)
    w1p = jnp.pad(W1, ((0, 0), (0, 56)))
    asd1 = jnp.pad(jnp.stack([a_s1, a_d1], axis=1), ((0, 56), (0, 0)))
    dumb = jnp.zeros((1, IDIM), f32)
    dumm = jnp.zeros((NP0, IDIM), f32)
    agga, aggb = _gat_layer(NP0, gs, gd, hcat1, w1p, asd1, dumb, dumm, False)

    def mk_wp(w):
        wa = w[:100]
        wb = w[100:200]
        wpad = jnp.concatenate([
            wa, jnp.zeros((28, w.shape[1]), f32),
            wb, jnp.zeros((28, w.shape[1]), f32)], axis=0)
        return jnp.pad(wpad, ((0, 0), (0, 256 - w.shape[1])))

    def mk_bm(b, m, np_):
        ba, bb = _split_pad_cols(b[None, :])
        ma, mb = _split_pad_cols(_pad_rows(m, np_))
        return jnp.concatenate([ba, bb], 1), jnp.concatenate([ma, mb], 1)

    for (w, a_s, a_d, bprev, mprev) in (
            (W2, a_s2, a_d2, b1, masks[0]), (W3, a_s3, a_d3, b2, masks[1])):
        hcat = jnp.concatenate([agga, aggb], axis=1)
        wp = mk_wp(w)
        asd = jnp.pad(jnp.stack([a_s, a_d], axis=1), ((0, 56), (0, 0)))
        bcat, mcat = mk_bm(bprev, mprev, NP0)
        agga, aggb = _gat_layer(NP0, gs, gd, hcat, wp, asd, bcat, mcat, True)

    ba3, bb3 = _split_pad_cols(b3[None, :])
    ma3, mb3 = _split_pad_cols(_pad_rows(masks[2], NP0))
    wpool = mk_wp(jnp.concatenate([Wp_rel, Wp_root], axis=1))[:, :128]
    ha, hb, z12 = _h3z_call(agga, aggb, ba3, bb3, ma3, mb3, wpool)
    z1 = z12[:, 0]
    z2 = z12[:, 1]

    ps = jnp.pad(src, (0, EP - E))
    pd = jnp.pad(dst, (0, EP - E))
    em = (jnp.arange(EP) < E).astype(f32)
    rank_prev = jnp.arange(NP0, dtype=jnp.int32)
    kprev = N
    npprev = NP0
    bp = bp_rel.reshape(1, 1)

    nt = N
    np_ = NP0
    while True:
        k = int(math.ceil(0.5 * nt))
        kp = _rup(k + 1, 8)
        ps, pd, em, sg0, sg1 = _p1_call(np_, npprev, kprev, ps, pd, em,
                                        rank_prev, z1)
        score, ts = _score_call(sg0, sg1, z2.reshape(1, np_), bp, nt)
        s2d = score.reshape(np_ // 256, 256)
        rank = _rank_call(s2d)[:, 0]
        hsa, hsb, zs = _tsmul_call(ha, hb, jnp.stack([z1, z2], axis=1),
                                   ts.reshape(np_, 1))
        hoa, hob, z1o, z2o = _p3_call(np_, k, kp, hsa, hsb,
                                      zs[:, 0], zs[:, 1], rank)
        rank_prev, kprev, npprev = rank, k, np_
        nt = k
        if nt <= OLEN:
            out = jnp.concatenate([hoa[:k, :100], hob[:k, :100]], axis=1)
            return jnp.pad(out, ((0, OLEN - k), (0, 0)))
        np_ = _rup(k, 256)
        ha = _pad_rows(hoa[:k], np_)
        hb = _pad_rows(hob[:k], np_)
        z1 = jnp.pad(z1o[:k], (0, np_ - k))
        z2 = jnp.pad(z2o[:k], (0, np_ - k))


# speedup vs baseline: 1.0139x; 1.0139x over previous
import functools
import math

import jax
import jax.numpy as jnp
from jax import lax
from jax.experimental import pallas as pl
from jax.experimental.pallas import tpu as pltpu
from jax.experimental.pallas import tpu_sc as plsc

N = 10000
E = 320000
IDIM = 128
HDIM = 200
ODIM = 200
OLEN = 512

NB = 128
HALF = 128
DP = 2 * HALF

NP0 = 10240
E2 = E + N
E2P = 331776
EP = 323584


def _rup(x, m):
    return (x + m - 1) // m * m


def _mm_body(prologue, hcat_ref, w_ref, asd_ref, b_ref, m_ref,
             xpa_ref, xpb_ref, ssd_ref, bm_ref):
    h = hcat_ref[...]
    if prologue:
        h = jnp.maximum(h + b_ref[...], 0.0) * m_ref[...]
    res = jnp.dot(h, w_ref[...], preferred_element_type=jnp.float32)
    z12 = jnp.zeros((res.shape[0], 28), jnp.float32)
    xpa_ref[...] = jnp.concatenate([res[:, :100], z12], axis=1)
    xpb_ref[...] = jnp.concatenate([res[:, 100:200], z12], axis=1)
    ssd = jnp.dot(res, asd_ref[...], preferred_element_type=jnp.float32)
    ssd_ref[...] = ssd
    bm_ref[...] = jnp.max(ssd[:, :1], axis=0, keepdims=True)[None]


def _mm_call(hcat, w, asd, b, m, prologue):
    np_, dpin = hcat.shape
    grid = np_ // 256
    return pl.pallas_call(
        functools.partial(_mm_body, prologue),
        grid=(grid,),
        in_specs=[
            pl.BlockSpec((256, dpin), lambda i: (i, 0)),
            pl.BlockSpec((dpin, 256), lambda i: (0, 0)),
            pl.BlockSpec((256, 2), lambda i: (0, 0)),
            pl.BlockSpec((1, dpin), lambda i: (0, 0)),
            pl.BlockSpec((256, dpin), lambda i: (i, 0)),
        ],
        out_specs=[
            pl.BlockSpec((256, HALF), lambda i: (i, 0)),
            pl.BlockSpec((256, HALF), lambda i: (i, 0)),
            pl.BlockSpec((256, 2), lambda i: (i, 0)),
            pl.BlockSpec((1, 1, 1), lambda i: (i, 0, 0)),
        ],
        out_shape=[
            jax.ShapeDtypeStruct((np_, HALF), jnp.float32),
            jax.ShapeDtypeStruct((np_, HALF), jnp.float32),
            jax.ShapeDtypeStruct((np_, 2), jnp.float32),
            jax.ShapeDtypeStruct((grid, 1, 1), jnp.float32),
        ],
    )(hcat, w, asd, b, m)


def _lrelu(v):
    return jnp.where(v >= 0, v, 0.2 * v)


def _fin1_body(t0_ref, t1_ref, ssd_ref, c_ref, out_ref):
    t = t0_ref[0, :] + t1_ref[0, :]
    t = jnp.maximum(t, 1e-30)
    out_ref[...] = _lrelu(jnp.log(t) + c_ref[0, 0] + ssd_ref[:, 1])[None, :]


def _fin1_call(t0, t1, ssd, c):
    np_ = ssd.shape[0]
    return pl.pallas_call(
        _fin1_body,
        out_shape=jax.ShapeDtypeStruct((1, np_), jnp.float32),
    )(t0.reshape(1, np_), t1.reshape(1, np_), ssd, c)


def _fin2_body(d0_ref, d1_ref, out_ref):
    out_ref[...] = 1.0 / (d0_ref[0, :] + d1_ref[0, :] + 1e-16)[None, :]


def _fin2_call(d0, d1):
    np_ = d0.shape[0]
    return pl.pallas_call(
        _fin2_body,
        out_shape=jax.ShapeDtypeStruct((1, np_), jnp.float32),
    )(d0.reshape(1, np_), d1.reshape(1, np_))


def _h3z_body(agga_ref, aggb_ref, ba_ref, bb_ref, ma_ref, mb_ref, wp_ref,
              ha_ref, hb_ref, z_ref):
    ha = jnp.maximum(agga_ref[...] + ba_ref[...], 0.0) * ma_ref[...]
    hb = jnp.maximum(aggb_ref[...] + bb_ref[...], 0.0) * mb_ref[...]
    ha_ref[...] = ha
    hb_ref[...] = hb
    hcat = jnp.concatenate([ha, hb], axis=1)
    z = jnp.dot(hcat, wp_ref[...], preferred_element_type=jnp.float32)
    z_ref[...] = z[:, :2]


def _h3z_call(agga, aggb, ba, bb, ma, mb, wp):
    np_ = agga.shape[0]
    grid = np_ // 256
    return pl.pallas_call(
        _h3z_body,
        grid=(grid,),
        in_specs=[
            pl.BlockSpec((256, HALF), lambda i: (i, 0)),
            pl.BlockSpec((256, HALF), lambda i: (i, 0)),
            pl.BlockSpec((1, HALF), lambda i: (0, 0)),
            pl.BlockSpec((1, HALF), lambda i: (0, 0)),
            pl.BlockSpec((256, HALF), lambda i: (i, 0)),
            pl.BlockSpec((256, HALF), lambda i: (i, 0)),
            pl.BlockSpec((DP, 128), lambda i: (0, 0)),
        ],
        out_specs=[
            pl.BlockSpec((256, HALF), lambda i: (i, 0)),
            pl.BlockSpec((256, HALF), lambda i: (i, 0)),
            pl.BlockSpec((256, 2), lambda i: (i, 0)),
        ],
        out_shape=[
            jax.ShapeDtypeStruct((np_, HALF), jnp.float32),
            jax.ShapeDtypeStruct((np_, HALF), jnp.float32),
            jax.ShapeDtypeStruct((np_, 2), jnp.float32),
        ],
    )(agga, aggb, ba, bb, ma, mb, wp)


def _score_body(nt, sagg0_ref, sagg1_ref, z2_ref, bp_ref, score_ref, ts_ref):
    np_ = z2_ref.shape[1]
    s = sagg0_ref[0, :] + sagg1_ref[0, :] + bp_ref[0, 0] + z2_ref[0, :]
    col = lax.broadcasted_iota(jnp.int32, (1, np_), 1)
    s = jnp.where(col < nt, s[None, :], -jnp.inf)
    score_ref[...] = s
    ts_ref[...] = jnp.tanh(s)


def _score_call(sagg0, sagg1, z2, bp, nt):
    np_ = z2.shape[1]
    return pl.pallas_call(
        functools.partial(_score_body, nt),
        out_shape=[
            jax.ShapeDtypeStruct((1, np_), jnp.float32),
            jax.ShapeDtypeStruct((1, np_), jnp.float32),
        ],
    )(sagg0.reshape(1, np_), sagg1.reshape(1, np_), z2, bp)


def _rank_body(s2d_ref, rank_ref):
    i = pl.program_id(0)
    nj = s2d_ref.shape[0]
    si = jnp.reshape(s2d_ref[i, :], (256, 1))
    ii = i * 256 + lax.broadcasted_iota(jnp.int32, (256, 1), 0)

    def jloop(j, acc):
        sj = jnp.reshape(s2d_ref[j, :], (1, 256))
        jj = j * 256 + lax.broadcasted_iota(jnp.int32, (1, 256), 1)
        gt = (sj > si).astype(jnp.float32)
        eq = jnp.logical_and(sj == si, jj < ii).astype(jnp.float32)
        return acc + jnp.sum(gt + eq, axis=1, keepdims=True)

    acc = lax.fori_loop(0, nj, jloop, jnp.zeros((256, 1), jnp.float32))
    rank_ref[...] = acc.astype(jnp.int32)


def _rank_call(s2d):
    np_ = s2d.shape[0] * 256
    return pl.pallas_call(
        _rank_body,
        grid=(s2d.shape[0],),
        in_specs=[pl.BlockSpec((s2d.shape[0], 256), lambda i: (0, 0))],
        out_specs=pl.BlockSpec((256, 1), lambda i: (i, 0)),
        out_shape=jax.ShapeDtypeStruct((np_, 1), jnp.int32),
    )(s2d)


_MESH_CACHE = []


_SC_PARAMS = pltpu.CompilerParams(needs_layout_passes=False)


def _mesh():
    if not _MESH_CACHE:
        _MESH_CACHE.append(plsc.VectorSubcoreMesh(
            core_axis_name="c", subcore_axis_name="s"))
    return _MESH_CACHE[0]


def _wid():
    c = lax.axis_index("c")
    s = lax.axis_index("s")
    return c, s, s * 2 + c


def _zero_acc1(acc, zb, sid, pw):
    def zi(i, _):
        zb[pl.ds(i * 16, 16)] = jnp.zeros((16,), jnp.float32)
        return 0
    lax.fori_loop(0, pw // 16, zi, 0)
    pltpu.sync_copy(zb, acc.at[pl.ds(sid * pw, pw)])


def _wb_acc1(acc, out, sid, pw, zb):
    pltpu.sync_copy(acc.at[pl.ds(sid * pw, pw)], zb)
    pltpu.sync_copy(zb, out.at[pl.ds(sid * pw, pw)])


def _s1_body(np_, gs, gd, ss, c_h, out0, out1, idx_s, idx_d, val, tbl, c_v, acc, zb):
    c, sid, w = _wid()
    pw = np_ // 16
    _zero_acc1(acc, zb, sid, pw)
    pltpu.sync_copy(ss, tbl)
    pltpu.sync_copy(c_h, c_v)
    plsc.subcore_barrier()
    epw = E2P // 32
    base0 = w * epw
    cmax = c_v[pl.ds(0, 16)][0]

    def chunk(j, _):
        base = base0 + j * NB
        pltpu.sync_copy(gs.at[pl.ds(base, NB)], idx_s)
        pltpu.sync_copy(gd.at[pl.ds(base, NB)], idx_d)

        def lane(v, _):
            sv = idx_s[pl.ds(v * 16, 16)]
            ssv = plsc.load_gather(tbl, [sv])
            eid = base + v * 16 + lax.iota(jnp.int32, 16)
            val[pl.ds(v * 16, 16)] = jnp.where(
                eid < E2, jnp.exp(ssv - cmax), 0.0)
            return 0

        lax.fori_loop(0, NB // 16, lane, 0)
        pltpu.sync_copy(val, acc.at[idx_d], add=True)
        return 0

    lax.fori_loop(0, epw // NB, chunk, 0)
    plsc.subcore_barrier()

    @pl.when(c == 0)
    def _():
        _wb_acc1(acc, out0, sid, pw, zb)

    @pl.when(c == 1)
    def _():
        _wb_acc1(acc, out1, sid, pw, zb)


def _s1_call(np_, gs, gd, ss, c_h):
    fn = pl.kernel(
        functools.partial(_s1_body, np_),
        out_type=[jax.ShapeDtypeStruct((np_,), jnp.float32),
                  jax.ShapeDtypeStruct((np_,), jnp.float32)],
        mesh=_mesh(),
        compiler_params=_SC_PARAMS,
        scratch_types=[
            pltpu.VMEM((NB,), jnp.int32),
            pltpu.VMEM((NB,), jnp.int32),
            pltpu.VMEM((NB,), jnp.float32),
            pltpu.VMEM((np_,), jnp.float32),
            pltpu.VMEM((16,), jnp.float32),
            pltpu.VMEM_SHARED((np_,), jnp.float32),
            pltpu.VMEM((np_ // 16,), jnp.float32),
        ],
    )
    return fn(gs, gd, ss, c_h)


def _s2_body(np_, gs, gd, ss, sd, mt, d_out0, d_out1, ex_out,
             idx_s, idx_d, val, tbl_s, tbl_d, tbl_m, acc, zb):
    c, sid, w = _wid()
    pw = np_ // 16
    _zero_acc1(acc, zb, sid, pw)
    pltpu.sync_copy(ss, tbl_s)
    pltpu.sync_copy(sd, tbl_d)
    pltpu.sync_copy(mt, tbl_m)
    plsc.subcore_barrier()
    epw = E2P // 32
    base0 = w * epw

    def chunk(j, _):
        base = base0 + j * NB
        pltpu.sync_copy(gs.at[pl.ds(base, NB)], idx_s)
        pltpu.sync_copy(gd.at[pl.ds(base, NB)], idx_d)

        def lane(v, _):
            sv = idx_s[pl.ds(v * 16, 16)]
            dv = idx_d[pl.ds(v * 16, 16)]
            al = _lrelu(plsc.load_gather(tbl_s, [sv]) +
                        plsc.load_gather(tbl_d, [dv]))
            mv = plsc.load_gather(tbl_m, [dv])
            eid = base + v * 16 + lax.iota(jnp.int32, 16)
            val[pl.ds(v * 16, 16)] = jnp.where(
                eid < E2, jnp.exp(al - mv), 0.0)
            return 0

        lax.fori_loop(0, NB // 16, lane, 0)
        pltpu.sync_copy(val, ex_out.at[pl.ds(base, NB)])
        pltpu.sync_copy(val, acc.at[idx_d], add=True)
        return 0

    lax.fori_loop(0, epw // NB, chunk, 0)
    plsc.subcore_barrier()

    @pl.when(c == 0)
    def _():
        _wb_acc1(acc, d_out0, sid, pw, zb)

    @pl.when(c == 1)
    def _():
        _wb_acc1(acc, d_out1, sid, pw, zb)


def _s2_call(np_, gs, gd, ss, sd, mt):
    fn = pl.kernel(
        functools.partial(_s2_body, np_),
        out_type=[
            jax.ShapeDtypeStruct((np_,), jnp.float32),
            jax.ShapeDtypeStruct((np_,), jnp.float32),
            jax.ShapeDtypeStruct((E2P,), jnp.float32),
        ],
        mesh=_mesh(),
        compiler_params=_SC_PARAMS,
        scratch_types=[
            pltpu.VMEM((NB,), jnp.int32),
            pltpu.VMEM((NB,), jnp.int32),
            pltpu.VMEM((NB,), jnp.float32),
            pltpu.VMEM((np_,), jnp.float32),
            pltpu.VMEM((np_,), jnp.float32),
            pltpu.VMEM((np_,), jnp.float32),
            pltpu.VMEM_SHARED((np_,), jnp.float32),
            pltpu.VMEM((np_ // 16,), jnp.float32),
        ],
    )
    return fn(gs, gd, ss, sd, mt)


def _msg_body(np_, gs, gd, ex, invd, xpa, xpb, agga, aggb,
              idx_s, idx_d, exv, cfv, rows, tbl_i, zb, acc, sem):
    c, sid, _ = _wid()
    prow = np_ // 16

    def zi(r, _):
        def zf(f, _):
            zb[r, pl.ds(f * 16, 16)] = jnp.zeros((16,), jnp.float32)
            return 0
        lax.fori_loop(0, 8, zf, 0)
        return 0
    lax.fori_loop(0, 16, zi, 0)

    def zcp(j, _):
        pltpu.sync_copy(zb, acc.at[pl.ds(sid * prow + j * 16, 16), :])
        return 0
    lax.fori_loop(0, prow // 16, zcp, 0)
    pltpu.sync_copy(invd, tbl_i)
    plsc.subcore_barrier()

    epw = E2P // 16
    base0 = sid * epw

    def run(xp, agg):
        def chunk(j, _):
            base = base0 + j * NB
            pltpu.sync_copy(gs.at[pl.ds(base, NB)], idx_s)
            pltpu.sync_copy(gd.at[pl.ds(base, NB)], idx_d)
            pltpu.sync_copy(ex.at[pl.ds(base, NB)], exv)
            cp = pltpu.async_copy(xp.at[idx_s], rows, sem)

            def lane(v, _):
                dv = idx_d[pl.ds(v * 16, 16)]
                iv = plsc.load_gather(tbl_i, [dv])
                cfv[pl.ds(v * 16, 16)] = exv[pl.ds(v * 16, 16)] * iv
                return 0

            lax.fori_loop(0, NB // 16, lane, 0)
            cp.wait()

            def scale(g, _):
                cf16 = cfv[pl.ds(g * 16, 16)]
                for l in range(16):
                    r = g * 16 + l
                    cs = cf16[l]

                    def sf(f, _):
                        rows[r, pl.ds(f * 16, 16)] = (
                            rows[r, pl.ds(f * 16, 16)] * cs)
                        return 0
                    lax.fori_loop(0, 8, sf, 0)
                return 0

            lax.fori_loop(0, NB // 16, scale, 0)
            pltpu.sync_copy(rows, acc.at[idx_d], add=True)
            return 0

        lax.fori_loop(0, epw // NB, chunk, 0)
        plsc.subcore_barrier()
        pltpu.sync_copy(acc.at[pl.ds(sid * prow, prow), :],
                        agg.at[pl.ds(sid * prow, prow), :])

    @pl.when(c == 0)
    def _():
        run(xpa, agga)

    @pl.when(c == 1)
    def _():
        run(xpb, aggb)


def _msg_call(np_, gs, gd, ex, invd, xpa, xpb):
    fn = pl.kernel(
        functools.partial(_msg_body, np_),
        out_type=[
            jax.ShapeDtypeStruct((np_, HALF), jnp.float32),
            jax.ShapeDtypeStruct((np_, HALF), jnp.float32),
        ],
        mesh=_mesh(),
        compiler_params=_SC_PARAMS,
        scratch_types=[
            pltpu.VMEM((NB,), jnp.int32),
            pltpu.VMEM((NB,), jnp.int32),
            pltpu.VMEM((NB,), jnp.float32),
            pltpu.VMEM((NB,), jnp.float32),
            pltpu.VMEM((NB, HALF), jnp.float32),
            pltpu.VMEM((np_,), jnp.float32),
            pltpu.VMEM((16, HALF), jnp.float32),
            pltpu.VMEM_SHARED((np_, HALF), jnp.float32),
            pltpu.SemaphoreType.DMA,
        ],
    )
    return fn(gs, gd, ex, invd, xpa, xpb)


def _p1_body(np_, npprev, kprev, ps, pd, em, rank, z1,
             ps_o, pd_o, em_o, sagg0, sagg1,
             idx_s, idx_d, emv, val, tbl_r, tbl_z, acc, zb):
    c, sid, w = _wid()
    pw = np_ // 16
    _zero_acc1(acc, zb, sid, pw)
    pltpu.sync_copy(rank, tbl_r)
    pltpu.sync_copy(z1, tbl_z)
    plsc.subcore_barrier()
    epw = EP // 32
    base0 = w * epw

    def chunk(j, _):
        base = base0 + j * NB
        pltpu.sync_copy(ps.at[pl.ds(base, NB)], idx_s)
        pltpu.sync_copy(pd.at[pl.ds(base, NB)], idx_d)
        pltpu.sync_copy(em.at[pl.ds(base, NB)], emv)

        def lane(v, _):
            sv = idx_s[pl.ds(v * 16, 16)]
            dv = idx_d[pl.ds(v * 16, 16)]
            rs = plsc.load_gather(tbl_r, [sv])
            rd = plsc.load_gather(tbl_r, [dv])
            oks = rs < kprev
            okd = rd < kprev
            ps2 = jnp.where(oks, rs, 0)
            pd2 = jnp.where(okd, rd, 0)
            em2 = emv[pl.ds(v * 16, 16)] * jnp.where(
                jnp.logical_and(oks, okd), 1.0, 0.0)
            idx_s[pl.ds(v * 16, 16)] = ps2
            idx_d[pl.ds(v * 16, 16)] = pd2
            emv[pl.ds(v * 16, 16)] = em2
            val[pl.ds(v * 16, 16)] = plsc.load_gather(tbl_z, [ps2]) * em2
            return 0

        lax.fori_loop(0, NB // 16, lane, 0)
        pltpu.sync_copy(idx_s, ps_o.at[pl.ds(base, NB)])
        pltpu.sync_copy(idx_d, pd_o.at[pl.ds(base, NB)])
        pltpu.sync_copy(emv, em_o.at[pl.ds(base, NB)])
        pltpu.sync_copy(val, acc.at[idx_d], add=True)
        return 0

    lax.fori_loop(0, epw // NB, chunk, 0)
    plsc.subcore_barrier()

    @pl.when(c == 0)
    def _():
        _wb_acc1(acc, sagg0, sid, pw, zb)

    @pl.when(c == 1)
    def _():
        _wb_acc1(acc, sagg1, sid, pw, zb)


def _p1_call(np_, npprev, kprev, ps, pd, em, rank, z1):
    fn = pl.kernel(
        functools.partial(_p1_body, np_, npprev, kprev),
        out_type=[
            jax.ShapeDtypeStruct((EP,), jnp.int32),
            jax.ShapeDtypeStruct((EP,), jnp.int32),
            jax.ShapeDtypeStruct((EP,), jnp.float32),
            jax.ShapeDtypeStruct((np_,), jnp.float32),
            jax.ShapeDtypeStruct((np_,), jnp.float32),
        ],
        mesh=_mesh(),
        compiler_params=_SC_PARAMS,
        scratch_types=[
            pltpu.VMEM((NB,), jnp.int32),
            pltpu.VMEM((NB,), jnp.int32),
            pltpu.VMEM((NB,), jnp.float32),
            pltpu.VMEM((NB,), jnp.float32),
            pltpu.VMEM((npprev,), jnp.int32),
            pltpu.VMEM((np_,), jnp.float32),
            pltpu.VMEM_SHARED((np_,), jnp.float32),
            pltpu.VMEM((np_ // 16,), jnp.float32),
        ],
    )
    return fn(ps, pd, em, rank, z1)


def _p3_body(np_, k, kp, ha, hb, z1, z2, ts, rank,
             hoa, hob, z1o, z2o,
             rowsa, rowsb, tsv, rkv, tgt, zav, zbv, sema, semb):
    c, sid, w = _wid()
    nch = np_ // 16

    def chunk(jj, _):
        cid = w + jj * 32

        @pl.when(cid < nch)
        def _():
            base = cid * 16
            pltpu.sync_copy(ts.at[pl.ds(base, 16)], tsv)
            pltpu.sync_copy(rank.at[pl.ds(base, 16)], rkv)
            pltpu.sync_copy(z1.at[pl.ds(base, 16)], zav)
            pltpu.sync_copy(z2.at[pl.ds(base, 16)], zbv)
            cpa = pltpu.async_copy(ha.at[pl.ds(base, 16), :], rowsa, sema)
            cpb = pltpu.async_copy(hb.at[pl.ds(base, 16), :], rowsb, semb)
            rk = rkv[pl.ds(0, 16)]
            tgt[pl.ds(0, 16)] = jnp.where(rk < k, rk, kp - 1)
            ts16 = tsv[pl.ds(0, 16)]
            zav[pl.ds(0, 16)] = zav[pl.ds(0, 16)] * ts16
            zbv[pl.ds(0, 16)] = zbv[pl.ds(0, 16)] * ts16
            cpa.wait()
            cpb.wait()
            for r in range(16):
                cs = ts16[r]

                def sf(f, _):
                    rowsa[r, pl.ds(f * 16, 16)] = (
                        rowsa[r, pl.ds(f * 16, 16)] * cs)
                    rowsb[r, pl.ds(f * 16, 16)] = (
                        rowsb[r, pl.ds(f * 16, 16)] * cs)
                    return 0
                lax.fori_loop(0, 8, sf, 0)
            pltpu.sync_copy(rowsa, hoa.at[tgt])
            pltpu.sync_copy(rowsb, hob.at[tgt])
            pltpu.sync_copy(zav, z1o.at[tgt])
            pltpu.sync_copy(zbv, z2o.at[tgt])
        return 0

    lax.fori_loop(0, (nch + 31) // 32, chunk, 0)


def _p3_call(np_, k, kp, ha, hb, z1, z2, ts, rank):
    fn = pl.kernel(
        functools.partial(_p3_body, np_, k, kp),
        out_type=[
            jax.ShapeDtypeStruct((kp, HALF), jnp.float32),
            jax.ShapeDtypeStruct((kp, HALF), jnp.float32),
            jax.ShapeDtypeStruct((kp,), jnp.float32),
            jax.ShapeDtypeStruct((kp,), jnp.float32),
        ],
        mesh=_mesh(),
        compiler_params=_SC_PARAMS,
        scratch_types=[
            pltpu.VMEM((16, HALF), jnp.float32),
            pltpu.VMEM((16, HALF), jnp.float32),
            pltpu.VMEM((16,), jnp.float32),
            pltpu.VMEM((16,), jnp.int32),
            pltpu.VMEM((16,), jnp.int32),
            pltpu.VMEM((16,), jnp.float32),
            pltpu.VMEM((16,), jnp.float32),
            pltpu.SemaphoreType.DMA,
            pltpu.SemaphoreType.DMA,
        ],
    )
    return fn(ha, hb, z1, z2, ts, rank)


def _pad_rows(a, np_):
    return jnp.pad(a, ((0, np_ - a.shape[0]),) + ((0, 0),) * (a.ndim - 1))


def _split_pad_cols(v):
    za = jnp.pad(v[..., :100], ((0, 0),) * (v.ndim - 1) + ((0, 28),))
    zb = jnp.pad(v[..., 100:200], ((0, 0),) * (v.ndim - 1) + ((0, 28),))
    return za, zb


def _gat_layer(np_, gs, gd, hcat, w_p, asd_p, b_prev, m_prev, prologue):
    xpa, xpb, ssd, bm = _mm_call(hcat, w_p, asd_p, b_prev, m_prev, prologue)
    c = jnp.max(bm).reshape(1, 1)
    ss = ssd[:, 0]
    c16 = jnp.broadcast_to(jnp.max(bm), (16,))
    t0, t1 = _s1_call(np_, gs, gd, ss, c16)
    mt = _fin1_call(t0, t1, ssd, c)[0]
    d0, d1, ex = _s2_call(np_, gs, gd, ss, ssd[:, 1], mt)
    invd = _fin2_call(d0, d1)[0]
    agga, aggb = _msg_call(np_, gs, gd, ex, invd, xpa, xpb)
    return agga, aggb


def kernel(x, edge_index, W1, a_s1, a_d1, b1, W2, a_s2, a_d2, b2,
           W3, a_s3, a_d3, b3, Wp_rel, bp_rel, Wp_root):
    f32 = jnp.float32
    src = edge_index[0].astype(jnp.int32)
    dst = edge_index[1].astype(jnp.int32)
    loop = jnp.arange(N, dtype=jnp.int32)
    gs = jnp.pad(jnp.concatenate([src, loop]), (0, E2P - E2))
    gd = jnp.pad(jnp.concatenate([dst, loop]), (0, E2P - E2))

    dk = jax.random.key(42)
    masks = []
    for i in range(3):
        keep = jax.random.bernoulli(jax.random.fold_in(dk, i), 0.8, (N, 200))
        masks.append(keep.astype(f32) * 1.25)

    hcat1 = _pad_rows(x, NP0)
    w1p = jnp.pad(W1, ((0, 0), (0, 56)))
    asd1 = jnp.pad(jnp.stack([a_s1, a_d1], axis=1), ((0, 56), (0, 0)))
    dumb = jnp.zeros((1, IDIM), f32)
    dumm = jnp.zeros((NP0, IDIM), f32)
    agga, aggb = _gat_layer(NP0, gs, gd, hcat1, w1p, asd1, dumb, dumm, False)

    def mk_wp(w):
        wa = w[:100]
        wb = w[100:200]
        wpad = jnp.concatenate([
            wa, jnp.zeros((28, w.shape[1]), f32),
            wb, jnp.zeros((28, w.shape[1]), f32)], axis=0)
        return jnp.pad(wpad, ((0, 0), (0, 256 - w.shape[1])))

    def mk_bm(b, m, np_):
        ba, bb = _split_pad_cols(b[None, :])
        ma, mb = _split_pad_cols(_pad_rows(m, np_))
        return jnp.concatenate([ba, bb], 1), jnp.concatenate([ma, mb], 1)

    for (w, a_s, a_d, bprev, mprev) in (
            (W2, a_s2, a_d2, b1, masks[0]), (W3, a_s3, a_d3, b2, masks[1])):
        hcat = jnp.concatenate([agga, aggb], axis=1)
        wp = mk_wp(w)
        asd = jnp.pad(jnp.stack([a_s, a_d], axis=1), ((0, 56), (0, 0)))
        bcat, mcat = mk_bm(bprev, mprev, NP0)
        agga, aggb = _gat_layer(NP0, gs, gd, hcat, wp, asd, bcat, mcat, True)

    ba3, bb3 = _split_pad_cols(b3[None, :])
    ma3, mb3 = _split_pad_cols(_pad_rows(masks[2], NP0))
    wpool = mk_wp(jnp.concatenate([Wp_rel, Wp_root], axis=1))[:, :128]
    ha, hb, z12 = _h3z_call(agga, aggb, ba3, bb3, ma3, mb3, wpool)
    z1 = z12[:, 0]
    z2 = z12[:, 1]

    ps = jnp.pad(src, (0, EP - E))
    pd = jnp.pad(dst, (0, EP - E))
    em = (jnp.arange(EP) < E).astype(f32)
    rank_prev = jnp.arange(NP0, dtype=jnp.int32)
    kprev = N
    npprev = NP0
    bp = bp_rel.reshape(1, 1)

    nt = N
    np_ = NP0
    while True:
        k = int(math.ceil(0.5 * nt))
        kp = _rup(k + 1, 8)
        ps, pd, em, sg0, sg1 = _p1_call(np_, npprev, kprev, ps, pd, em,
                                        rank_prev, z1)
        score, ts = _score_call(sg0, sg1, z2.reshape(1, np_), bp, nt)
        s2d = score.reshape(np_ // 256, 256)
        rank = _rank_call(s2d)[:, 0]
        hoa, hob, z1o, z2o = _p3_call(np_, k, kp, ha, hb, z1, z2,
                                      ts[0], rank)
        rank_prev, kprev, npprev = rank, k, np_
        nt = k
        if nt <= OLEN:
            out = jnp.concatenate([hoa[:k, :100], hob[:k, :100]], axis=1)
            return jnp.pad(out, ((0, OLEN - k), (0, 0)))
        np_ = _rup(k, 256)
        ha = _pad_rows(hoa[:k], np_)
        hb = _pad_rows(hob[:k], np_)
        z1 = jnp.pad(z1o[:k], (0, np_ - k))
        z2 = jnp.pad(z2o[:k], (0, np_ - k))
